# baseline (speedup 1.0000x reference)
import functools

import jax
import jax.numpy as jnp
from jax import lax
from jax.experimental import pallas as pl
from jax.experimental.pallas import tpu as pltpu
from jax.experimental.pallas import tpu_sc as plsc

_NUM_TIMESTEPS = 1000
_BETA_START = 1e-4
_BETA_END = 0.02
_TAB_PAD = 1024


def _sc_gather_coeffs(tabs, t):
    B = t.shape[0]
    mesh = plsc.VectorSubcoreMesh(
        core_axis_name="c", subcore_axis_name="s", num_cores=1, num_subcores=1
    )

    @functools.partial(
        pl.kernel,
        out_type=jax.ShapeDtypeStruct((2 * B,), jnp.float32),
        mesh=mesh,
        compiler_params=pltpu.CompilerParams(needs_layout_passes=False),
        scratch_types=[
            pltpu.VMEM((2, _TAB_PAD), jnp.float32),
            pltpu.VMEM((B,), jnp.int32),
            pltpu.VMEM((2 * B,), jnp.float32),
            pltpu.SemaphoreType.DMA,
            pltpu.SemaphoreType.DMA,
        ],
    )
    def gather_kernel(tabs_hbm, t_hbm, c_out, tabs_v, t_v, c_v, sem1, sem2):
        cid = lax.axis_index("c")
        sid = lax.axis_index("s")

        @pl.when(jnp.logical_and(cid == 0, sid == 0))
        def _():
            cp1 = pltpu.make_async_copy(tabs_hbm, tabs_v, sem1)
            cp2 = pltpu.make_async_copy(t_hbm, t_v, sem2)
            cp1.start()
            cp2.start()
            cp1.wait()
            cp2.wait()
            for i in range(B // 16):
                idx = t_v[pl.ds(i * 16, 16)]
                c_v[pl.ds(i * 16, 16)] = plsc.load_gather(tabs_v, [jnp.zeros((16,), jnp.int32), idx])
                c_v[pl.ds(B + i * 16, 16)] = plsc.load_gather(tabs_v, [jnp.ones((16,), jnp.int32), idx])
            pltpu.sync_copy(c_v, c_out)

    return gather_kernel(tabs, t)


def _fma_body(c_ref, x_ref, n_ref, o_ref):
    i = pl.program_id(0)
    B = pl.num_programs(0)
    a = c_ref[i]
    b = c_ref[B + i]
    o_ref[...] = a * x_ref[...] + b * n_ref[...]


def kernel(x_start, t, noise):
    B, C, H, W = x_start.shape

    betas = jnp.linspace(_BETA_START, _BETA_END, _NUM_TIMESTEPS, dtype=jnp.float32)
    ac = jnp.cumprod(1.0 - betas, axis=0)
    pad = jnp.zeros((_TAB_PAD - _NUM_TIMESTEPS,), jnp.float32)
    tabs = jnp.stack(
        [
            jnp.concatenate([jnp.sqrt(ac), pad]),
            jnp.concatenate([jnp.sqrt(1.0 - ac), pad]),
        ]
    )

    coeffs = _sc_gather_coeffs(tabs, t)

    blk = (1, C, H, W)
    idx = lambda i: (i, 0, 0, 0)
    out = pl.pallas_call(
        _fma_body,
        grid=(B,),
        in_specs=[
            pl.BlockSpec(memory_space=pltpu.SMEM),
            pl.BlockSpec(blk, idx),
            pl.BlockSpec(blk, idx),
        ],
        out_specs=pl.BlockSpec(blk, idx),
        out_shape=jax.ShapeDtypeStruct((B, C, H, W), jnp.float32),
    )(coeffs, x_start, noise)
    return out


# speedup vs baseline: 1.0014x; 1.0014x over previous
import functools

import jax
import jax.numpy as jnp
from jax.experimental import pallas as pl
from jax.experimental.pallas import tpu as pltpu
from jax.experimental.pallas import tpu_sc as plsc

_NUM_TIMESTEPS = 1000
_BETA_START = 1e-4
_BETA_END = 0.02
_TAB_PAD = 1024


def _sc_gather_coeffs(tabs, t):
    B = t.shape[0]
    mesh = plsc.VectorSubcoreMesh(
        core_axis_name="c", subcore_axis_name="s", num_cores=1, num_subcores=1
    )

    @functools.partial(
        pl.kernel,
        out_type=jax.ShapeDtypeStruct((2 * B,), jnp.float32),
        mesh=mesh,
        compiler_params=pltpu.CompilerParams(needs_layout_passes=False),
        scratch_types=[
            pltpu.VMEM((2, _TAB_PAD), jnp.float32),
            pltpu.VMEM((B,), jnp.int32),
            pltpu.VMEM((2 * B,), jnp.float32),
            pltpu.SemaphoreType.DMA,
            pltpu.SemaphoreType.DMA,
        ],
    )
    def gather_kernel(tabs_hbm, t_hbm, c_out, tabs_v, t_v, c_v, sem1, sem2):
        cp1 = pltpu.make_async_copy(tabs_hbm, tabs_v, sem1)
        cp2 = pltpu.make_async_copy(t_hbm, t_v, sem2)
        cp1.start()
        cp2.start()
        cp1.wait()
        cp2.wait()
        for i in range(B // 16):
            idx = t_v[pl.ds(i * 16, 16)]
            c_v[pl.ds(i * 16, 16)] = plsc.load_gather(
                tabs_v, [jnp.zeros((16,), jnp.int32), idx]
            )
            c_v[pl.ds(B + i * 16, 16)] = plsc.load_gather(
                tabs_v, [jnp.ones((16,), jnp.int32), idx]
            )
        pltpu.sync_copy(c_v, c_out)

    return gather_kernel(tabs, t)


def _fma_body(c_ref, x_ref, n_ref, o_ref):
    i = pl.program_id(0)
    B = pl.num_programs(0)
    a = c_ref[i]
    b = c_ref[B + i]
    o_ref[...] = a * x_ref[...] + b * n_ref[...]


def kernel(x_start, t, noise):
    B, C, H, W = x_start.shape

    betas = jnp.linspace(_BETA_START, _BETA_END, _NUM_TIMESTEPS, dtype=jnp.float32)
    ac = jnp.cumprod(1.0 - betas, axis=0)
    pad = jnp.zeros((_TAB_PAD - _NUM_TIMESTEPS,), jnp.float32)
    tabs = jnp.stack(
        [
            jnp.concatenate([jnp.sqrt(ac), pad]),
            jnp.concatenate([jnp.sqrt(1.0 - ac), pad]),
        ]
    )

    coeffs = _sc_gather_coeffs(tabs, t)

    blk = (1, C, H, W)
    idx = lambda i: (i, 0, 0, 0)
    out = pl.pallas_call(
        _fma_body,
        grid=(B,),
        in_specs=[
            pl.BlockSpec(memory_space=pltpu.SMEM),
            pl.BlockSpec(blk, idx),
            pl.BlockSpec(blk, idx),
        ],
        out_specs=pl.BlockSpec(blk, idx),
        out_shape=jax.ShapeDtypeStruct((B, C, H, W), jnp.float32),
    )(coeffs, x_start, noise)
    return out


# speedup vs baseline: 1.0015x; 1.0001x over previous
import functools

import jax
import jax.numpy as jnp
from jax.experimental import pallas as pl
from jax.experimental.pallas import tpu as pltpu
from jax.experimental.pallas import tpu_sc as plsc

_NUM_TIMESTEPS = 1000
_BETA_START = 1e-4
_BETA_END = 0.02
_TAB_PAD = 1024


def _sc_gather_coeffs(tabs, t):
    B = t.shape[0]
    mesh = plsc.VectorSubcoreMesh(
        core_axis_name="c", subcore_axis_name="s", num_cores=1, num_subcores=1
    )

    @functools.partial(
        pl.kernel,
        out_type=jax.ShapeDtypeStruct((2 * B,), jnp.float32),
        mesh=mesh,
        compiler_params=pltpu.CompilerParams(
            needs_layout_passes=False, skip_device_barrier=True
        ),
        scratch_types=[
            pltpu.VMEM((2, _TAB_PAD), jnp.float32),
            pltpu.VMEM((B,), jnp.int32),
            pltpu.VMEM((2 * B,), jnp.float32),
            pltpu.SemaphoreType.DMA,
            pltpu.SemaphoreType.DMA,
        ],
    )
    def gather_kernel(tabs_hbm, t_hbm, c_out, tabs_v, t_v, c_v, sem1, sem2):
        cp1 = pltpu.make_async_copy(tabs_hbm, tabs_v, sem1)
        cp2 = pltpu.make_async_copy(t_hbm, t_v, sem2)
        cp1.start()
        cp2.start()
        cp1.wait()
        cp2.wait()
        for i in range(B // 16):
            idx = t_v[pl.ds(i * 16, 16)]
            c_v[pl.ds(i * 16, 16)] = plsc.load_gather(
                tabs_v, [jnp.zeros((16,), jnp.int32), idx]
            )
            c_v[pl.ds(B + i * 16, 16)] = plsc.load_gather(
                tabs_v, [jnp.ones((16,), jnp.int32), idx]
            )
        pltpu.sync_copy(c_v, c_out)

    return gather_kernel(tabs, t)


def _fma_body(c_ref, x_ref, n_ref, o_ref):
    i = pl.program_id(0)
    B = pl.num_programs(0)
    a = c_ref[i]
    b = c_ref[B + i]
    o_ref[...] = a * x_ref[...] + b * n_ref[...]


def kernel(x_start, t, noise):
    B, C, H, W = x_start.shape

    betas = jnp.linspace(_BETA_START, _BETA_END, _NUM_TIMESTEPS, dtype=jnp.float32)
    ac = jnp.cumprod(1.0 - betas, axis=0)
    pad = jnp.zeros((_TAB_PAD - _NUM_TIMESTEPS,), jnp.float32)
    tabs = jnp.stack(
        [
            jnp.concatenate([jnp.sqrt(ac), pad]),
            jnp.concatenate([jnp.sqrt(1.0 - ac), pad]),
        ]
    )

    coeffs = _sc_gather_coeffs(tabs, t)

    blk = (1, C, H, W)
    idx = lambda i: (i, 0, 0, 0)
    out = pl.pallas_call(
        _fma_body,
        grid=(B,),
        in_specs=[
            pl.BlockSpec(memory_space=pltpu.SMEM),
            pl.BlockSpec(blk, idx),
            pl.BlockSpec(blk, idx),
        ],
        out_specs=pl.BlockSpec(blk, idx),
        out_shape=jax.ShapeDtypeStruct((B, C, H, W), jnp.float32),
        compiler_params=pltpu.CompilerParams(skip_device_barrier=True),
    )(coeffs, x_start, noise)
    return out


# speedup vs baseline: 1.0019x; 1.0004x over previous
import functools

import jax
import jax.numpy as jnp
from jax.experimental import pallas as pl
from jax.experimental.pallas import tpu as pltpu
from jax.experimental.pallas import tpu_sc as plsc

_NUM_TIMESTEPS = 1000
_BETA_START = 1e-4
_BETA_END = 0.02
_TAB_PAD = 1024


def _sc_gather_coeffs(tabs, t):
    B = t.shape[0]
    mesh = plsc.VectorSubcoreMesh(
        core_axis_name="c", subcore_axis_name="s", num_cores=1, num_subcores=1
    )

    @functools.partial(
        pl.kernel,
        out_type=jax.ShapeDtypeStruct((2 * B,), jnp.float32),
        mesh=mesh,
        compiler_params=pltpu.CompilerParams(needs_layout_passes=False),
        scratch_types=[
            pltpu.VMEM((2, _TAB_PAD), jnp.float32),
            pltpu.VMEM((B,), jnp.int32),
            pltpu.VMEM((2 * B,), jnp.float32),
            pltpu.SemaphoreType.DMA,
            pltpu.SemaphoreType.DMA,
        ],
    )
    def gather_kernel(tabs_hbm, t_hbm, c_out, tabs_v, t_v, c_v, sem1, sem2):
        cp1 = pltpu.make_async_copy(tabs_hbm, tabs_v, sem1)
        cp2 = pltpu.make_async_copy(t_hbm, t_v, sem2)
        cp1.start()
        cp2.start()
        cp1.wait()
        cp2.wait()
        for i in range(B // 16):
            idx = t_v[pl.ds(i * 16, 16)]
            c_v[pl.ds(i * 16, 16)] = plsc.load_gather(
                tabs_v, [jnp.zeros((16,), jnp.int32), idx]
            )
            c_v[pl.ds(B + i * 16, 16)] = plsc.load_gather(
                tabs_v, [jnp.ones((16,), jnp.int32), idx]
            )
        pltpu.sync_copy(c_v, c_out)

    return gather_kernel(tabs, t)


def _fma_body(c_ref, x_ref, n_ref, o_ref):
    i = pl.program_id(0)
    B = pl.num_programs(0)
    a = c_ref[i]
    b = c_ref[B + i]
    o_ref[...] = a * x_ref[...] + b * n_ref[...]


def kernel(x_start, t, noise):
    B, C, H, W = x_start.shape

    betas = jnp.linspace(_BETA_START, _BETA_END, _NUM_TIMESTEPS, dtype=jnp.float32)
    ac = jnp.cumprod(1.0 - betas, axis=0)
    pad = jnp.zeros((_TAB_PAD - _NUM_TIMESTEPS,), jnp.float32)
    tabs = jnp.stack(
        [
            jnp.concatenate([jnp.sqrt(ac), pad]),
            jnp.concatenate([jnp.sqrt(1.0 - ac), pad]),
        ]
    )

    coeffs = _sc_gather_coeffs(tabs, t)

    blk = (1, C, H, W)
    idx = lambda i: (i, 0, 0, 0)
    out = pl.pallas_call(
        _fma_body,
        grid=(B,),
        in_specs=[
            pl.BlockSpec(memory_space=pltpu.SMEM),
            pl.BlockSpec(blk, idx),
            pl.BlockSpec(blk, idx),
        ],
        out_specs=pl.BlockSpec(blk, idx),
        out_shape=jax.ShapeDtypeStruct((B, C, H, W), jnp.float32),
    )(coeffs, x_start, noise)
    return out
